# baseline (speedup 1.0000x reference)
import jax
import jax.numpy as jnp
from jax import lax
from jax.experimental import pallas as pl


def _rvq_body(num_levels, num_codes, z_ref, cb_ref,
              zq_ref, c0_ref, c1_ref, c2_ref, c3_ref, loss_ref):
    pid = pl.program_id(0)
    t = z_ref.shape[0]
    inv_n = 1.0 / (zq_ref.shape[0] * zq_ref.shape[1] * pl.num_programs(0))
    code_refs = (c0_ref, c1_ref, c2_ref, c3_ref)

    z = z_ref[...]
    r = z
    zq = jnp.zeros_like(z)
    lane_iota = lax.broadcasted_iota(jnp.int32, (t, num_codes), 1)
    ones_row = jnp.ones((1, z.shape[1]), jnp.float32)

    for lvl in range(num_levels):
        cb = cb_ref[lvl]
        c2 = lax.dot_general(ones_row, cb * cb, (((1,), (1,)), ((), ())),
                             precision=lax.Precision.HIGHEST,
                             preferred_element_type=jnp.float32)
        mm = lax.dot_general(r, cb, (((1,), (1,)), ((), ())),
                             preferred_element_type=jnp.float32)
        d = c2 - 2.0 * mm
        m = jnp.min(d, axis=1, keepdims=True)
        idx = jnp.min(jnp.where(d == m, lane_iota, num_codes), axis=1)
        oh = (lane_iota == idx[:, None]).astype(jnp.float32)
        sel = lax.dot_general(oh, cb, (((1,), (0,)), ((), ())),
                              precision=lax.Precision.HIGHEST,
                              preferred_element_type=jnp.float32)
        zq = zq + sel
        r = r - sel
        code_refs[lvl][...] = idx[:, None]

    zq_ref[...] = z + (zq - z)

    @pl.when(pid == 0)
    def _init():
        loss_ref[...] = jnp.zeros((1, 1), jnp.float32)
    loss_ref[...] += (jnp.sum(r * r) * inv_n).reshape(1, 1)


def kernel(z, codebooks):
    b, d = z.shape
    num_levels, num_codes, _ = codebooks.shape
    t = 1024 if b % 1024 == 0 else b
    nblk = b // t

    def body(*refs):
        _rvq_body(num_levels, num_codes, *refs)

    out = pl.pallas_call(
        body,
        grid=(nblk,),
        in_specs=[
            pl.BlockSpec((t, d), lambda i: (i, 0)),
            pl.BlockSpec((num_levels, num_codes, d), lambda i: (0, 0, 0)),
        ],
        out_specs=[
            pl.BlockSpec((t, d), lambda i: (i, 0)),
            pl.BlockSpec((t, 1), lambda i: (i, 0)),
            pl.BlockSpec((t, 1), lambda i: (i, 0)),
            pl.BlockSpec((t, 1), lambda i: (i, 0)),
            pl.BlockSpec((t, 1), lambda i: (i, 0)),
            pl.BlockSpec((1, 1), lambda i: (0, 0)),
        ],
        out_shape=[
            jax.ShapeDtypeStruct((b, d), jnp.float32),
            jax.ShapeDtypeStruct((b, 1), jnp.int32),
            jax.ShapeDtypeStruct((b, 1), jnp.int32),
            jax.ShapeDtypeStruct((b, 1), jnp.int32),
            jax.ShapeDtypeStruct((b, 1), jnp.int32),
            jax.ShapeDtypeStruct((1, 1), jnp.float32),
        ],
    )(z, codebooks)

    zq, c0, c1, c2, c3, loss = out
    codes = jnp.concatenate([c0, c1, c2, c3], axis=1)
    return (zq, codes, loss.reshape(()))
